# baseline (speedup 1.0000x reference)
import jax
import jax.numpy as jnp
from jax import lax
from jax.experimental import pallas as pl
from jax.experimental.pallas import tpu as pltpu
from jax.experimental.pallas import tpu_sc as plsc

MEM = 10000
B = 2048
NCL = 100
D = 3 * 32 * 32
PK = 128
NC = 2
NS = 16
NT = NC * NS
GR = 16
NGROUPS = MEM // GR
CHUNK = 16
LMAX = B + NT * CHUNK


def _extract(vmem64, j):
    return vmem64[pl.ds(j, 1)][0]


def _body(mw, ms, mp, sw, ss, bp_in, li_l, lm_l, meta,
          ow, os_, op_,
          bw, bs, bp, idxb, idxm, vmeta, rsem, wsem):
    c = lax.axis_index("c")
    s = lax.axis_index("s")
    w = s * NC + c

    SLAB = MEM // NT
    r0 = w * SLAB
    cps = [
        pltpu.async_copy(mw.at[pl.ds(r0, SLAB)], ow.at[pl.ds(r0, SLAB)], wsem),
        pltpu.async_copy(ms.at[pl.ds(r0, SLAB)], os_.at[pl.ds(r0, SLAB)], wsem),
        pltpu.async_copy(mp.at[pl.ds(r0, SLAB)], op_.at[pl.ds(r0, SLAB)], wsem),
    ]

    @pl.when(w == 0)
    def _rem():
        rr = NT * SLAB
        rn = MEM - NT * SLAB
        for cp in (
            pltpu.async_copy(mw.at[pl.ds(rr, rn)], ow.at[pl.ds(rr, rn)], wsem),
            pltpu.async_copy(ms.at[pl.ds(rr, rn)], os_.at[pl.ds(rr, rn)], wsem),
            pltpu.async_copy(mp.at[pl.ds(rr, rn)], op_.at[pl.ds(rr, rn)], wsem),
        ):
            cp.wait()

    for cp in cps:
        cp.wait()

    pltpu.sync_copy(meta, vmeta)
    start_e = _extract(vmeta, w)
    nch = _extract(vmeta, NT + w)

    def p2(i, carry):
        e0 = pl.multiple_of(start_e + i * CHUNK, CHUNK)
        pltpu.sync_copy(li_l.at[pl.ds(e0, CHUNK)], idxb)
        pltpu.sync_copy(lm_l.at[pl.ds(e0, CHUNK)], idxm)
        gts = (
            pltpu.async_copy(sw.at[idxb], bw, rsem),
            pltpu.async_copy(ss.at[idxb], bs, rsem),
            pltpu.async_copy(bp_in.at[idxb], bp, rsem),
        )
        for cp in gts:
            cp.wait()
        sts = (
            pltpu.async_copy(bw, ow.at[idxm], wsem),
            pltpu.async_copy(bs, os_.at[idxm], wsem),
            pltpu.async_copy(bp, op_.at[idxm], wsem),
        )
        for cp in sts:
            cp.wait()
        return carry

    lax.fori_loop(0, nch, p2, 0)


def kernel(mem_weak, mem_strong, mem_label, mem_partial, mem_task, mem_index,
           sample_weak, sample_strong, label, partial, task, index, rand_idx):
    i32 = jnp.int32
    f32 = jnp.float32
    mw2 = mem_weak.reshape(MEM, D)
    ms2 = mem_strong.reshape(MEM, D)
    sw2 = sample_weak.reshape(B, D)
    ss2 = sample_strong.reshape(B, D)

    mp_pad = jnp.concatenate(
        [mem_partial, mem_label[:, None], mem_task[:, None],
         mem_index[:, None], jnp.zeros((MEM, PK - NCL - 3), i32)], axis=1)
    taskcol = jnp.full((B,), task, i32)
    bp_pad = jnp.concatenate(
        [partial, label[:, None], taskcol[:, None],
         index[:, None], jnp.zeros((B, PK - NCL - 3), i32)], axis=1)

    ii = jnp.arange(B, dtype=i32)
    win = jnp.full((MEM,), -1, i32).at[rand_idx].max(ii, mode="drop")
    safe = jnp.where(rand_idx < MEM, rand_idx, 0)
    is_win = (rand_idx < MEM) & (win[safe] == ii)
    slab = MEM // NT
    row_owner = jnp.where(rand_idx >= NT * slab, 0, rand_idx // slab)
    owner = jnp.where(is_win, row_owner, NT).astype(i32)
    ordr = jnp.argsort(owner, stable=True)
    si = ii[ordr]
    sm = jnp.where(is_win, rand_idx, 0).astype(i32)[ordr]
    cnt = jnp.bincount(owner, length=NT + 1)[:NT].astype(i32)
    cntp = ((cnt + CHUNK - 1) // CHUNK) * CHUNK
    zero1 = jnp.zeros((1,), i32)
    start = jnp.concatenate([zero1, jnp.cumsum(cntp)[:-1].astype(i32)])
    rawstart = jnp.concatenate([zero1, jnp.cumsum(cnt)[:-1].astype(i32)])
    pos = jnp.arange(LMAX, dtype=i32)
    bkt = jnp.searchsorted(start, pos, side="right").astype(i32) - 1
    off = pos - start[bkt]
    src = rawstart[bkt] + jnp.minimum(off, jnp.maximum(cnt[bkt] - 1, 0))
    src = jnp.clip(src, 0, B - 1)
    li_l = si[src]
    lm_l = sm[src]
    meta = jnp.concatenate([start, cntp // CHUNK]).astype(i32)

    ow, os_, op_ = pl.kernel(
        _body,
        out_type=[
            jax.ShapeDtypeStruct((MEM, D), f32),
            jax.ShapeDtypeStruct((MEM, D), f32),
            jax.ShapeDtypeStruct((MEM, PK), i32),
        ],
        mesh=plsc.VectorSubcoreMesh(core_axis_name="c", subcore_axis_name="s"),
        scratch_types=[
            pltpu.VMEM((GR, D), f32),
            pltpu.VMEM((GR, D), f32),
            pltpu.VMEM((GR, PK), i32),
            pltpu.VMEM((CHUNK,), i32),
            pltpu.VMEM((CHUNK,), i32),
            pltpu.VMEM((64,), i32),
            pltpu.SemaphoreType.DMA,
            pltpu.SemaphoreType.DMA,
        ],
    )(mw2, ms2, mp_pad, sw2, ss2, bp_pad, li_l, lm_l, meta)

    return (ow.reshape(mem_weak.shape), os_.reshape(mem_strong.shape),
            op_[:, NCL], op_[:, :NCL], op_[:, NCL + 1], op_[:, NCL + 2])


# speedup vs baseline: 7.4445x; 7.4445x over previous
import jax
import jax.numpy as jnp
from jax import lax
from jax.experimental import pallas as pl
from jax.experimental.pallas import tpu as pltpu
from jax.experimental.pallas import tpu_sc as plsc

MEM = 10000
B = 2048
NCL = 100
D = 3 * 32 * 32
PK = 128
NC = 2
NS = 16
NT = NC * NS
GR = 8
NGROUPS = MEM // GR
CHUNK = 8
LMAX = B + NT * CHUNK


def _extract(vmem64, j):
    return vmem64[pl.ds(j, 1)][0]


def _body(mw, ms, mp, sw, ss, bp_in, li_l, lm_l, meta,
          ow, os_, op_,
          bw0, bs0, bp0, bw1, bs1, bp1, idxb, idxm, vmeta,
          rsem0, rsem1, wsem):
    c = lax.axis_index("c")
    s = lax.axis_index("s")
    w = s * NC + c

    ng = (NGROUPS - w + NT - 1) // NT

    def rbase(i):
        return (i * NT + w) * GR

    def rd_descs(i, bufs, sem):
        r0 = rbase(i)
        return (
            pltpu.make_async_copy(mw.at[pl.ds(r0, GR)], bufs[0], sem),
            pltpu.make_async_copy(ms.at[pl.ds(r0, GR)], bufs[1], sem),
            pltpu.make_async_copy(mp.at[pl.ds(r0, GR)], bufs[2], sem),
        )

    def wr_descs(i, bufs):
        r0 = rbase(i)
        return (
            pltpu.make_async_copy(bufs[0], ow.at[pl.ds(r0, GR)], wsem),
            pltpu.make_async_copy(bufs[1], os_.at[pl.ds(r0, GR)], wsem),
            pltpu.make_async_copy(bufs[2], op_.at[pl.ds(r0, GR)], wsem),
        )

    set0 = (bw0, bs0, bp0)
    set1 = (bw1, bs1, bp1)

    def run(i, cur, nxt, rcur, rnxt):
        @pl.when(i > 0)
        def _():
            for d in wr_descs(i - 1, nxt):
                d.wait()

        @pl.when(i + 1 < ng)
        def _():
            for d in rd_descs(i + 1, nxt, rnxt):
                d.start()

        for d in rd_descs(i, cur, rcur):
            d.wait()
        for d in wr_descs(i, cur):
            d.start()

    def p1(i, carry):
        even = (i % 2) == 0

        @pl.when(even)
        def _():
            run(i, set0, set1, rsem0, rsem1)

        @pl.when(jnp.logical_not(even))
        def _():
            run(i, set1, set0, rsem1, rsem0)

        return carry

    @pl.when(ng > 0)
    def _():
        for d in rd_descs(0, set0, rsem0):
            d.start()

    lax.fori_loop(0, ng, p1, 0)

    @pl.when(ng > 0)
    def _():
        for d in wr_descs(ng - 1, set0):
            d.wait()

    pltpu.sync_copy(meta, vmeta)
    start_e = _extract(vmeta, w)
    nch = _extract(vmeta, NT + w)

    def p2(i, carry):
        e0 = pl.multiple_of(start_e + i * CHUNK, CHUNK)
        pltpu.sync_copy(li_l.at[pl.ds(e0, CHUNK)], idxb)
        pltpu.sync_copy(lm_l.at[pl.ds(e0, CHUNK)], idxm)
        gts = (
            pltpu.async_copy(sw.at[idxb], bw0, rsem0),
            pltpu.async_copy(ss.at[idxb], bs0, rsem0),
            pltpu.async_copy(bp_in.at[idxb], bp0, rsem0),
        )
        for cp in gts:
            cp.wait()
        sts = (
            pltpu.async_copy(bw0, ow.at[idxm], wsem),
            pltpu.async_copy(bs0, os_.at[idxm], wsem),
            pltpu.async_copy(bp0, op_.at[idxm], wsem),
        )
        for cp in sts:
            cp.wait()
        return carry

    lax.fori_loop(0, nch, p2, 0)


def kernel(mem_weak, mem_strong, mem_label, mem_partial, mem_task, mem_index,
           sample_weak, sample_strong, label, partial, task, index, rand_idx):
    i32 = jnp.int32
    f32 = jnp.float32
    mw2 = mem_weak.reshape(MEM, D)
    ms2 = mem_strong.reshape(MEM, D)
    sw2 = sample_weak.reshape(B, D)
    ss2 = sample_strong.reshape(B, D)

    mp_pad = jnp.concatenate(
        [mem_partial, mem_label[:, None], mem_task[:, None],
         mem_index[:, None], jnp.zeros((MEM, PK - NCL - 3), i32)], axis=1)
    taskcol = jnp.full((B,), task, i32)
    bp_pad = jnp.concatenate(
        [partial, label[:, None], taskcol[:, None],
         index[:, None], jnp.zeros((B, PK - NCL - 3), i32)], axis=1)

    ii = jnp.arange(B, dtype=i32)
    win = jnp.full((MEM,), -1, i32).at[rand_idx].max(ii, mode="drop")
    safe = jnp.where(rand_idx < MEM, rand_idx, 0)
    is_win = (rand_idx < MEM) & (win[safe] == ii)
    row_owner = (rand_idx // GR) % NT
    owner = jnp.where(is_win, row_owner, NT).astype(i32)
    ordr = jnp.argsort(owner, stable=True)
    si = ii[ordr]
    sm = jnp.where(is_win, rand_idx, 0).astype(i32)[ordr]
    cnt = jnp.bincount(owner, length=NT + 1)[:NT].astype(i32)
    cntp = ((cnt + CHUNK - 1) // CHUNK) * CHUNK
    zero1 = jnp.zeros((1,), i32)
    start = jnp.concatenate([zero1, jnp.cumsum(cntp)[:-1].astype(i32)])
    rawstart = jnp.concatenate([zero1, jnp.cumsum(cnt)[:-1].astype(i32)])
    pos = jnp.arange(LMAX, dtype=i32)
    bkt = jnp.searchsorted(start, pos, side="right").astype(i32) - 1
    off = pos - start[bkt]
    src = rawstart[bkt] + jnp.minimum(off, jnp.maximum(cnt[bkt] - 1, 0))
    src = jnp.clip(src, 0, B - 1)
    li_l = si[src]
    lm_l = sm[src]
    meta = jnp.concatenate([start, cntp // CHUNK]).astype(i32)

    ow, os_, op_ = pl.kernel(
        _body,
        out_type=[
            jax.ShapeDtypeStruct((MEM, D), f32),
            jax.ShapeDtypeStruct((MEM, D), f32),
            jax.ShapeDtypeStruct((MEM, PK), i32),
        ],
        mesh=plsc.VectorSubcoreMesh(core_axis_name="c", subcore_axis_name="s"),
        scratch_types=[
            pltpu.VMEM((GR, D), f32),
            pltpu.VMEM((GR, D), f32),
            pltpu.VMEM((GR, PK), i32),
            pltpu.VMEM((GR, D), f32),
            pltpu.VMEM((GR, D), f32),
            pltpu.VMEM((GR, PK), i32),
            pltpu.VMEM((CHUNK,), i32),
            pltpu.VMEM((CHUNK,), i32),
            pltpu.VMEM((64,), i32),
            pltpu.SemaphoreType.DMA,
            pltpu.SemaphoreType.DMA,
            pltpu.SemaphoreType.DMA,
        ],
    )(mw2, ms2, mp_pad, sw2, ss2, bp_pad, li_l, lm_l, meta)

    return (ow.reshape(mem_weak.shape), os_.reshape(mem_strong.shape),
            op_[:, NCL], op_[:, :NCL], op_[:, NCL + 1], op_[:, NCL + 2])


# speedup vs baseline: 8.5389x; 1.1470x over previous
import jax
import jax.numpy as jnp
from jax import lax
from jax.experimental import pallas as pl
from jax.experimental.pallas import tpu as pltpu
from jax.experimental.pallas import tpu_sc as plsc

MEM = 10000
B = 2048
NCL = 100
D = 3 * 32 * 32
PK = 128
NC = 2
NS = 16
NT = NC * NS
GR = 16
NGROUPS = MEM // GR
NB = B // 16


def _lane(vec, k):
    return vec[k]


def _body(mw, ms, mp, sw, ss, bp_in, rand_hbm,
          ow, os_, op_,
          bw, bs, bp, rv, cm, cb, tmp, idxb, idxm, rsem, wsem):
    cax = lax.axis_index("c")
    sax = lax.axis_index("s")
    w = sax * NC + cax

    ng = (NGROUPS - w + NT - 1) // NT

    def p1(i, carry):
        r0 = (i * NT + w) * GR
        rds = (
            pltpu.async_copy(mw.at[pl.ds(r0, GR)], bw, rsem),
            pltpu.async_copy(ms.at[pl.ds(r0, GR)], bs, rsem),
            pltpu.async_copy(mp.at[pl.ds(r0, GR)], bp, rsem),
        )
        for cp in rds:
            cp.wait()
        wrs = (
            pltpu.async_copy(bw, ow.at[pl.ds(r0, GR)], wsem),
            pltpu.async_copy(bs, os_.at[pl.ds(r0, GR)], wsem),
            pltpu.async_copy(bp, op_.at[pl.ds(r0, GR)], wsem),
        )
        for cp in wrs:
            cp.wait()
        return carry

    lax.fori_loop(0, ng, p1, 0)

    pltpu.sync_copy(rand_hbm, rv)
    li = lax.iota(jnp.int32, 16)

    def apply_chunk():
        gts = (
            pltpu.async_copy(sw.at[idxb], bw, rsem),
            pltpu.async_copy(ss.at[idxb], bs, rsem),
            pltpu.async_copy(bp_in.at[idxb], bp, rsem),
        )
        for cp in gts:
            cp.wait()
        sts = (
            pltpu.async_copy(bw, ow.at[idxm], wsem),
            pltpu.async_copy(bs, os_.at[idxm], wsem),
            pltpu.async_copy(bp, op_.at[idxm], wsem),
        )
        for cp in sts:
            cp.wait()

    def p2(ci, carry):
        base = pl.multiple_of(ci * 16, 16)
        r = rv[pl.ds(base, 16)]
        hit = (r < MEM) & (((r >> 4) & (NT - 1)) == w)
        nh = _lane(plsc.all_reduce_population_count(hit), 0)

        @pl.when(nh > 0)
        def _heavy():
            bvec = li + ci * 16
            plsc.store_compressed(cm.at[...], r, mask=hit)
            plsc.store_compressed(cb.at[...], bvec, mask=hit)
            cmr = cm[...]
            cbr = cb[...]
            m0 = _lane(cmr, 0)
            b0 = _lane(cbr, 0)
            cmv = jnp.where(li < nh, cmr, m0)
            cbv = jnp.where(li < nh, cbr, b0)
            dup = li < 0
            for k in range(15):
                dup = dup | ((cmv == (li * 0 + _lane(cmr, k))) & (li > k)
                             & (li < nh) & (k < nh))
            ndup = _lane(plsc.all_reduce_population_count(dup), 0)

            @pl.when(ndup == 0)
            def _fast():
                idxm[...] = cmv
                idxb[...] = cbv
                apply_chunk()

            @pl.when(ndup > 0)
            def _fallback():
                for k in range(16):
                    @pl.when(k < nh)
                    def _one(k=k):
                        idxm[...] = li * 0 + _lane(cmr, k)
                        idxb[...] = li * 0 + _lane(cbr, k)
                        apply_chunk()

        return carry

    lax.fori_loop(0, NB, p2, 0)


def kernel(mem_weak, mem_strong, mem_label, mem_partial, mem_task, mem_index,
           sample_weak, sample_strong, label, partial, task, index, rand_idx):
    i32 = jnp.int32
    f32 = jnp.float32
    mw2 = mem_weak.reshape(MEM, D)
    ms2 = mem_strong.reshape(MEM, D)
    sw2 = sample_weak.reshape(B, D)
    ss2 = sample_strong.reshape(B, D)

    mp_pad = jnp.concatenate(
        [mem_partial, mem_label[:, None], mem_task[:, None],
         mem_index[:, None], jnp.zeros((MEM, PK - NCL - 3), i32)], axis=1)
    taskcol = jnp.full((B,), task, i32)
    bp_pad = jnp.concatenate(
        [partial, label[:, None], taskcol[:, None],
         index[:, None], jnp.zeros((B, PK - NCL - 3), i32)], axis=1)

    ow, os_, op_ = pl.kernel(
        _body,
        out_type=[
            jax.ShapeDtypeStruct((MEM, D), f32),
            jax.ShapeDtypeStruct((MEM, D), f32),
            jax.ShapeDtypeStruct((MEM, PK), i32),
        ],
        mesh=plsc.VectorSubcoreMesh(core_axis_name="c", subcore_axis_name="s"),
        compiler_params=pltpu.CompilerParams(needs_layout_passes=False),
        scratch_types=[
            pltpu.VMEM((GR, D), f32),
            pltpu.VMEM((GR, D), f32),
            pltpu.VMEM((GR, PK), i32),
            pltpu.VMEM((B,), i32),
            pltpu.VMEM((16,), i32),
            pltpu.VMEM((16,), i32),
            pltpu.VMEM((16,), i32),
            pltpu.VMEM((16,), i32),
            pltpu.VMEM((16,), i32),
            pltpu.SemaphoreType.DMA,
            pltpu.SemaphoreType.DMA,
        ],
    )(mw2, ms2, mp_pad, sw2, ss2, bp_pad, rand_idx)

    return (ow.reshape(mem_weak.shape), os_.reshape(mem_strong.shape),
            op_[:, NCL], op_[:, :NCL], op_[:, NCL + 1], op_[:, NCL + 2])


# speedup vs baseline: 9.1477x; 1.0713x over previous
import jax
import jax.numpy as jnp
from jax import lax
from jax.experimental import pallas as pl
from jax.experimental.pallas import tpu as pltpu
from jax.experimental.pallas import tpu_sc as plsc

MEM = 10000
B = 2048
NCL = 100
D = 3 * 32 * 32
PK = 128
NC = 2
NS = 16
NT = NC * NS
GR = 16
NGROUPS = MEM // GR
NB = B // 16


def _lane(vec, k):
    return vec[k]


def _body(mw, ms, mp, sw, ss, bp_in, rand_hbm,
          ow, os_, op_,
          bw, bs, bp, rv, pm, pb, idxb, idxm, rsem, wsem):
    cax = lax.axis_index("c")
    sax = lax.axis_index("s")
    w = sax * NC + cax

    ng = (NGROUPS - w + NT - 1) // NT

    def p1(i, carry):
        r0 = (i * NT + w) * GR
        rds = (
            pltpu.async_copy(mw.at[pl.ds(r0, GR)], bw, rsem),
            pltpu.async_copy(ms.at[pl.ds(r0, GR)], bs, rsem),
            pltpu.async_copy(mp.at[pl.ds(r0, GR)], bp, rsem),
        )
        for cp in rds:
            cp.wait()
        wrs = (
            pltpu.async_copy(bw, ow.at[pl.ds(r0, GR)], wsem),
            pltpu.async_copy(bs, os_.at[pl.ds(r0, GR)], wsem),
            pltpu.async_copy(bp, op_.at[pl.ds(r0, GR)], wsem),
        )
        for cp in wrs:
            cp.wait()
        return carry

    lax.fori_loop(0, ng, p1, 0)

    pltpu.sync_copy(rand_hbm, rv)
    li = lax.iota(jnp.int32, 16)

    def apply_chunk():
        gts = (
            pltpu.async_copy(sw.at[idxb], bw, rsem),
            pltpu.async_copy(ss.at[idxb], bs, rsem),
            pltpu.async_copy(bp_in.at[idxb], bp, rsem),
        )
        for cp in gts:
            cp.wait()
        sts = (
            pltpu.async_copy(bw, ow.at[idxm], wsem),
            pltpu.async_copy(bs, os_.at[idxm], wsem),
            pltpu.async_copy(bp, op_.at[idxm], wsem),
        )
        for cp in sts:
            cp.wait()

    def scan(ci, cnt):
        base = pl.multiple_of(ci * 16, 16)
        r = rv[pl.ds(base, 16)]
        hit = (r < MEM) & (((r >> 4) & (NT - 1)) == w)
        h32 = jnp.where(hit, 1, 0)
        nh = _lane(plsc.all_reduce_population_count(hit), 0)

        @pl.when(nh > 0)
        def _append():
            rank = li * 0
            for k in range(15):
                rank = rank + jnp.where((li > k) & (_lane(h32, k) > 0), 1, 0)
            plsc.store_scatter(pm.at[...], [cnt + rank], r, mask=hit)
            plsc.store_scatter(pb.at[...], [cnt + rank], li + ci * 16,
                               mask=hit)

        return cnt + nh

    cnt = lax.fori_loop(0, NB, scan, jnp.int32(0))
    nchunks = (cnt + 15) // 16

    def apply(t, carry):
        o = pl.multiple_of(t * 16, 16)
        mv = pm[pl.ds(o, 16)]
        bv = pb[pl.ds(o, 16)]
        vc = jnp.minimum(cnt - o, 16)
        mvp = jnp.where(li < vc, mv, _lane(mv, 0))
        bvp = jnp.where(li < vc, bv, _lane(bv, 0))
        dup = li < 0
        for k in range(15):
            dup = dup | ((mvp == (li * 0 + _lane(mv, k))) & (li > k)
                         & (li < vc) & (k < vc))
        ndup = _lane(plsc.all_reduce_population_count(dup), 0)

        @pl.when(ndup == 0)
        def _fast():
            idxm[...] = mvp
            idxb[...] = bvp
            apply_chunk()

        @pl.when(ndup > 0)
        def _fallback():
            for k in range(16):
                @pl.when(k < vc)
                def _one(k=k):
                    idxm[...] = li * 0 + _lane(mvp, k)
                    idxb[...] = li * 0 + _lane(bvp, k)
                    apply_chunk()

        return carry

    lax.fori_loop(0, nchunks, apply, 0)


def kernel(mem_weak, mem_strong, mem_label, mem_partial, mem_task, mem_index,
           sample_weak, sample_strong, label, partial, task, index, rand_idx):
    i32 = jnp.int32
    f32 = jnp.float32
    mw2 = mem_weak.reshape(MEM, D)
    ms2 = mem_strong.reshape(MEM, D)
    sw2 = sample_weak.reshape(B, D)
    ss2 = sample_strong.reshape(B, D)

    mp_pad = jnp.concatenate(
        [mem_partial, mem_label[:, None], mem_task[:, None],
         mem_index[:, None], jnp.zeros((MEM, PK - NCL - 3), i32)], axis=1)
    taskcol = jnp.full((B,), task, i32)
    bp_pad = jnp.concatenate(
        [partial, label[:, None], taskcol[:, None],
         index[:, None], jnp.zeros((B, PK - NCL - 3), i32)], axis=1)

    ow, os_, op_ = pl.kernel(
        _body,
        out_type=[
            jax.ShapeDtypeStruct((MEM, D), f32),
            jax.ShapeDtypeStruct((MEM, D), f32),
            jax.ShapeDtypeStruct((MEM, PK), i32),
        ],
        mesh=plsc.VectorSubcoreMesh(core_axis_name="c", subcore_axis_name="s"),
        compiler_params=pltpu.CompilerParams(needs_layout_passes=False),
        scratch_types=[
            pltpu.VMEM((GR, D), f32),
            pltpu.VMEM((GR, D), f32),
            pltpu.VMEM((GR, PK), i32),
            pltpu.VMEM((B,), i32),
            pltpu.VMEM((B + 16,), i32),
            pltpu.VMEM((B + 16,), i32),
            pltpu.VMEM((16,), i32),
            pltpu.VMEM((16,), i32),
            pltpu.SemaphoreType.DMA,
            pltpu.SemaphoreType.DMA,
        ],
    )(mw2, ms2, mp_pad, sw2, ss2, bp_pad, rand_idx)

    return (ow.reshape(mem_weak.shape), os_.reshape(mem_strong.shape),
            op_[:, NCL], op_[:, :NCL], op_[:, NCL + 1], op_[:, NCL + 2])
